# baseline (speedup 1.0000x reference)
import functools

import jax
import jax.numpy as jnp
from jax import lax
from jax.experimental import pallas as pl
from jax.experimental.pallas import tpu as pltpu
from jax.experimental.pallas import tpu_sc as plsc

N_NODES = 10000
D_FEAT = 128
N_EDGES = 320000

_NC = 2
_NS = 16
_NW = _NC * _NS
_EPW = N_EDGES // _NW
_B = 80
_CHUNKS = _EPW // _B
_L = 16
_NBUF = 3

_mesh = plsc.VectorSubcoreMesh(core_axis_name="c", subcore_axis_name="s")


def _norms_tc(z_ref, o_ref):
    zz = z_ref[...]
    o_ref[...] = jnp.sum(zz * zz, axis=1)


def _make_norms(z):
    return pl.pallas_call(
        _norms_tc,
        out_shape=jax.ShapeDtypeStruct((N_NODES,), jnp.float32),
    )(z)


@functools.partial(
    pl.kernel,
    out_type=jax.ShapeDtypeStruct((N_EDGES,), jnp.float32),
    mesh=_mesh,
    scratch_types=[
        pltpu.VMEM((_CHUNKS, _B), jnp.int32),
        pltpu.VMEM((_CHUNKS, _B), jnp.int32),
        pltpu.VMEM((_NBUF, _B, D_FEAT), jnp.float32),
        pltpu.VMEM((N_NODES,), jnp.float32),
        pltpu.VMEM((_EPW,), jnp.float32),
        pltpu.SemaphoreType.DMA((_NBUF,)),
        pltpu.SemaphoreType.DMA((_NBUF,)),
    ],
    compiler_params=pltpu.CompilerParams(needs_layout_passes=False),
)
def _gae_sc(z_hbm, src_hbm, dst_hbm, norm_hbm, out_hbm,
            src_v, dst_v, rows_s, norm_v, out_v, semr, sema):
    wid = lax.axis_index("s") * _NC + lax.axis_index("c")

    pltpu.sync_copy(src_hbm.at[wid], src_v)
    pltpu.sync_copy(dst_hbm.at[wid], dst_v)
    pltpu.sync_copy(norm_hbm, norm_v)

    lane = lax.iota(jnp.int32, _L)

    def _start_rows(c):
        b = c % _NBUF
        pltpu.async_copy(z_hbm.at[src_v.at[c]], rows_s.at[b], semr.at[b])

    def _wait_rows(c):
        b = c % _NBUF
        pltpu.make_async_copy(z_hbm.at[src_v.at[c]], rows_s.at[b],
                              semr.at[b]).wait()

    def _start_add(c):
        b = c % _NBUF
        pltpu.async_copy(z_hbm.at[dst_v.at[c]], rows_s.at[b],
                         sema.at[b], add=True)

    def _wait_add(c):
        b = c % _NBUF
        pltpu.make_async_copy(z_hbm.at[dst_v.at[c]], rows_s.at[b],
                              sema.at[b]).wait()

    _start_rows(0)
    _start_rows(1)
    _wait_rows(0)
    _start_add(0)

    def chunk_body(c, carry):
        b = c % _NBUF

        @pl.when(c + 2 < _CHUNKS)
        def _():
            _start_rows(c + 2)

        @pl.when(c + 1 < _CHUNKS)
        def _():
            _wait_rows(c + 1)
            _start_add(c + 1)

        _wait_add(c)

        def group_body(g, carry2):
            row_idx = g * _L + lane
            _K = 8

            def d_body(d, carry3):
                accs, cols = carry3
                new_accs = []
                new_cols = []
                for k in range(_K):
                    s = plsc.load_gather(rows_s.at[b], [row_idx, cols[k]])
                    new_accs.append(accs[k] + s * s)
                    new_cols.append((cols[k] + _K) & (D_FEAT - 1))
                return tuple(new_accs), tuple(new_cols)

            zero = jnp.zeros((_L,), jnp.float32)
            accs, _ = lax.fori_loop(
                0, D_FEAT // _K, d_body,
                ((zero,) * _K,
                 tuple(lane + k for k in range(_K))),
                unroll=4)
            while len(accs) > 1:
                accs = tuple(accs[i] + accs[i + 1]
                             for i in range(0, len(accs), 2))
            q = accs[0]
            sidx = src_v[c, pl.ds(g * _L, _L)]
            didx = dst_v[c, pl.ds(g * _L, _L)]
            nsum = (plsc.load_gather(norm_v, [sidx])
                    + plsc.load_gather(norm_v, [didx]))
            dots = (q - nsum) * 0.5
            out_v[pl.ds(c * _B + g * _L, _L)] = 1.0 / (1.0 + jnp.exp(-dots))
            return carry2

        lax.fori_loop(0, _B // _L, group_body, 0)
        return carry

    lax.fori_loop(0, _CHUNKS, chunk_body, 0)
    pltpu.sync_copy(out_v, out_hbm.at[pl.ds(wid * _EPW, _EPW)])


def kernel(z, edge_index):
    ei = edge_index.astype(jnp.int32)
    src = ei[0].reshape(_NW, _CHUNKS, _B)
    dst = ei[1].reshape(_NW, _CHUNKS, _B)
    norms = _make_norms(z)
    return _gae_sc(z, src, dst, norms)


# speedup vs baseline: 1.0149x; 1.0149x over previous
import functools

import jax
import jax.numpy as jnp
from jax import lax
from jax.experimental import pallas as pl
from jax.experimental.pallas import tpu as pltpu
from jax.experimental.pallas import tpu_sc as plsc

N_NODES = 10000
D_FEAT = 128
N_EDGES = 320000

_NC = 2
_NS = 16
_NW = _NC * _NS
_EPW = N_EDGES // _NW
_B = 80
_CHUNKS = _EPW // _B
_L = 16
_NBUF = 5

_mesh = plsc.VectorSubcoreMesh(core_axis_name="c", subcore_axis_name="s")


def _norms_tc(z_ref, o_ref):
    zz = z_ref[...]
    o_ref[...] = jnp.sum(zz * zz, axis=1)


def _make_norms(z):
    return pl.pallas_call(
        _norms_tc,
        out_shape=jax.ShapeDtypeStruct((N_NODES,), jnp.float32),
    )(z)


@functools.partial(
    pl.kernel,
    out_type=jax.ShapeDtypeStruct((N_EDGES,), jnp.float32),
    mesh=_mesh,
    scratch_types=[
        pltpu.VMEM((_CHUNKS, _B), jnp.int32),
        pltpu.VMEM((_CHUNKS, _B), jnp.int32),
        pltpu.VMEM((_NBUF, _B, D_FEAT), jnp.float32),
        pltpu.VMEM((N_NODES,), jnp.float32),
        pltpu.VMEM((_EPW,), jnp.float32),
        pltpu.SemaphoreType.DMA((_NBUF,)),
        pltpu.SemaphoreType.DMA((_NBUF,)),
    ],
    compiler_params=pltpu.CompilerParams(needs_layout_passes=False),
)
def _gae_sc(z_hbm, src_hbm, dst_hbm, norm_hbm, out_hbm,
            src_v, dst_v, rows_s, norm_v, out_v, semr, sema):
    wid = lax.axis_index("s") * _NC + lax.axis_index("c")

    pltpu.sync_copy(src_hbm.at[wid], src_v)
    pltpu.sync_copy(dst_hbm.at[wid], dst_v)
    pltpu.sync_copy(norm_hbm, norm_v)

    lane = lax.iota(jnp.int32, _L)

    def _start_rows(c):
        b = c % _NBUF
        pltpu.async_copy(z_hbm.at[src_v.at[c]], rows_s.at[b], semr.at[b])

    def _wait_rows(c):
        b = c % _NBUF
        pltpu.make_async_copy(z_hbm.at[src_v.at[c]], rows_s.at[b],
                              semr.at[b]).wait()

    def _start_add(c):
        b = c % _NBUF
        pltpu.async_copy(z_hbm.at[dst_v.at[c]], rows_s.at[b],
                         sema.at[b], add=True)

    def _wait_add(c):
        b = c % _NBUF
        pltpu.make_async_copy(z_hbm.at[dst_v.at[c]], rows_s.at[b],
                              sema.at[b]).wait()

    for i in range(_NBUF - 1):
        _start_rows(i)
    _wait_rows(0)
    _start_add(0)

    def chunk_body(c, carry):
        b = c % _NBUF

        @pl.when(c + _NBUF - 1 < _CHUNKS)
        def _():
            _start_rows(c + _NBUF - 1)

        @pl.when(c + 1 < _CHUNKS)
        def _():
            _wait_rows(c + 1)
            _start_add(c + 1)

        _wait_add(c)

        def group_body(g, carry2):
            row_idx = g * _L + lane
            _K = 8

            def d_body(d, carry3):
                accs, cols = carry3
                new_accs = []
                new_cols = []
                for k in range(_K):
                    s = plsc.load_gather(rows_s.at[b], [row_idx, cols[k]])
                    new_accs.append(accs[k] + s * s)
                    new_cols.append((cols[k] + _K) & (D_FEAT - 1))
                return tuple(new_accs), tuple(new_cols)

            zero = jnp.zeros((_L,), jnp.float32)
            accs, _ = lax.fori_loop(
                0, D_FEAT // _K, d_body,
                ((zero,) * _K,
                 tuple(lane + k for k in range(_K))),
                unroll=4)
            while len(accs) > 1:
                accs = tuple(accs[i] + accs[i + 1]
                             for i in range(0, len(accs), 2))
            q = accs[0]
            sidx = src_v[c, pl.ds(g * _L, _L)]
            didx = dst_v[c, pl.ds(g * _L, _L)]
            nsum = (plsc.load_gather(norm_v, [sidx])
                    + plsc.load_gather(norm_v, [didx]))
            dots = (q - nsum) * 0.5
            out_v[pl.ds(c * _B + g * _L, _L)] = 1.0 / (1.0 + jnp.exp(-dots))
            return carry2

        lax.fori_loop(0, _B // _L, group_body, 0)
        return carry

    lax.fori_loop(0, _CHUNKS, chunk_body, 0)
    pltpu.sync_copy(out_v, out_hbm.at[pl.ds(wid * _EPW, _EPW)])


def kernel(z, edge_index):
    ei = edge_index.astype(jnp.int32)
    src = ei[0].reshape(_NW, _CHUNKS, _B)
    dst = ei[1].reshape(_NW, _CHUNKS, _B)
    norms = _make_norms(z)
    return _gae_sc(z, src, dst, norms)


# speedup vs baseline: 1.1839x; 1.1665x over previous
import functools

import jax
import jax.numpy as jnp
from jax import lax
from jax.experimental import pallas as pl
from jax.experimental.pallas import tpu as pltpu
from jax.experimental.pallas import tpu_sc as plsc

N_NODES = 10000
D_FEAT = 128
N_EDGES = 320000

_NC = 2
_NS = 16
_NW = _NC * _NS
_EPW = N_EDGES // _NW
_B = 80
_CHUNKS = _EPW // _B
_L = 16
_NBUF = 6
_AA = 3

_mesh = plsc.VectorSubcoreMesh(core_axis_name="c", subcore_axis_name="s")


def _norms_tc(z_ref, o_ref):
    zz = z_ref[...]
    o_ref[...] = jnp.sum(zz * zz, axis=1)


def _make_norms(z):
    return pl.pallas_call(
        _norms_tc,
        out_shape=jax.ShapeDtypeStruct((N_NODES,), jnp.float32),
    )(z)


@functools.partial(
    pl.kernel,
    out_type=jax.ShapeDtypeStruct((N_EDGES,), jnp.float32),
    mesh=_mesh,
    scratch_types=[
        pltpu.VMEM((_CHUNKS, _B), jnp.int32),
        pltpu.VMEM((_CHUNKS, _B), jnp.int32),
        pltpu.VMEM((_NBUF, _B, D_FEAT), jnp.float32),
        pltpu.VMEM((N_NODES,), jnp.float32),
        pltpu.VMEM((_EPW,), jnp.float32),
        pltpu.SemaphoreType.DMA((_NBUF,)),
        pltpu.SemaphoreType.DMA((_NBUF,)),
    ],
    compiler_params=pltpu.CompilerParams(needs_layout_passes=False),
)
def _gae_sc(z_hbm, src_hbm, dst_hbm, norm_hbm, out_hbm,
            src_v, dst_v, rows_s, norm_v, out_v, semr, sema):
    wid = lax.axis_index("s") * _NC + lax.axis_index("c")

    pltpu.sync_copy(src_hbm.at[wid], src_v)
    pltpu.sync_copy(dst_hbm.at[wid], dst_v)
    pltpu.sync_copy(norm_hbm, norm_v)

    lane = lax.iota(jnp.int32, _L)

    def _start_rows(c):
        b = c % _NBUF
        pltpu.async_copy(z_hbm.at[src_v.at[c]], rows_s.at[b], semr.at[b])

    def _wait_rows(c):
        b = c % _NBUF
        pltpu.make_async_copy(z_hbm.at[src_v.at[c]], rows_s.at[b],
                              semr.at[b]).wait()

    def _start_add(c):
        b = c % _NBUF
        pltpu.async_copy(z_hbm.at[dst_v.at[c]], rows_s.at[b],
                         sema.at[b], add=True)

    def _wait_add(c):
        b = c % _NBUF
        pltpu.make_async_copy(z_hbm.at[dst_v.at[c]], rows_s.at[b],
                              sema.at[b]).wait()

    for i in range(_NBUF - 1):
        _start_rows(i)
    for j in range(_AA):
        _wait_rows(j)
        _start_add(j)

    def chunk_body(c, carry):
        b = c % _NBUF

        @pl.when(c + _NBUF - 1 < _CHUNKS)
        def _():
            _start_rows(c + _NBUF - 1)

        @pl.when(c + _AA < _CHUNKS)
        def _():
            _wait_rows(c + _AA)
            _start_add(c + _AA)

        _wait_add(c)

        def group_body(g, carry2):
            row_idx = g * _L + lane
            _K = 8

            def d_body(d, carry3):
                accs, cols = carry3
                new_accs = []
                new_cols = []
                for k in range(_K):
                    s = plsc.load_gather(rows_s.at[b], [row_idx, cols[k]])
                    new_accs.append(accs[k] + s * s)
                    new_cols.append((cols[k] + _K) & (D_FEAT - 1))
                return tuple(new_accs), tuple(new_cols)

            zero = jnp.zeros((_L,), jnp.float32)
            accs, _ = lax.fori_loop(
                0, D_FEAT // _K, d_body,
                ((zero,) * _K,
                 tuple(lane + k for k in range(_K))),
                unroll=4)
            while len(accs) > 1:
                accs = tuple(accs[i] + accs[i + 1]
                             for i in range(0, len(accs), 2))
            q = accs[0]
            sidx = src_v[c, pl.ds(g * _L, _L)]
            didx = dst_v[c, pl.ds(g * _L, _L)]
            nsum = (plsc.load_gather(norm_v, [sidx])
                    + plsc.load_gather(norm_v, [didx]))
            dots = (q - nsum) * 0.5
            out_v[pl.ds(c * _B + g * _L, _L)] = 1.0 / (1.0 + jnp.exp(-dots))
            return carry2

        lax.fori_loop(0, _B // _L, group_body, 0)
        return carry

    lax.fori_loop(0, _CHUNKS, chunk_body, 0)
    pltpu.sync_copy(out_v, out_hbm.at[pl.ds(wid * _EPW, _EPW)])


def kernel(z, edge_index):
    ei = edge_index.astype(jnp.int32)
    src = ei[0].reshape(_NW, _CHUNKS, _B)
    dst = ei[1].reshape(_NW, _CHUNKS, _B)
    norms = _make_norms(z)
    return _gae_sc(z, src, dst, norms)
